# baseline (device time: 70108 ns/iter reference)
import jax
import jax.numpy as jnp
from jax import lax
from jax.experimental import pallas as pl
from jax.experimental.pallas import tpu as pltpu

N_DEV = 8
SQ = 512
D = 1024
DH = 128
H_LOC = 8
G_LOC = 2
SKV = 2048
SCALE = 0.08838834764831843
SCALE_LOG2E = SCALE * 1.4426950408889634
VAUG = 256
CHUNK = SQ // N_DEV
BLK = 128
NBLK = SQ // BLK


def _body(
    x_ref, wq_ref, wo_ref, kx_ref, vx_ref, out_ref,
    xb_ref, wqb_ref, wob_ref, kf_ref, vf_ref, kb_ref, vb_ref,
    stage_ref, comm_ref, gbuf_ref, pblk_ref,
    kv_sems, rs_send_sems, rs_recv_sems, ag_send_sems, ag_recv_sems,
):
    my = lax.axis_index("i")

    kcopy = pltpu.make_async_copy(
        kx_ref.at[:, pl.ds(2 * my, G_LOC), :], kf_ref, kv_sems.at[0])
    vcopy = pltpu.make_async_copy(
        vx_ref.at[:, pl.ds(2 * my, G_LOC), :], vf_ref, kv_sems.at[1])
    kcopy.start()
    vcopy.start()

    xb_ref[...] = x_ref[...].astype(jnp.bfloat16)
    wqb_ref[...] = wq_ref[...].astype(jnp.bfloat16)
    wob_ref[...] = wo_ref[...].astype(jnp.bfloat16)
    kcopy.wait()
    kb_ref[...] = kf_ref[...].astype(jnp.bfloat16)
    vcopy.wait()
    vb_ref[:, :, :DH] = vf_ref[...].astype(jnp.bfloat16)
    lane = lax.broadcasted_iota(jnp.int32, (SKV, G_LOC, VAUG - DH), 2)
    vb_ref[:, :, DH:] = jnp.where(lane == 0, 1.0, 0.0).astype(jnp.bfloat16)

    barrier_sem = pltpu.get_barrier_semaphore()
    for o in range(1, N_DEV):
        pl.semaphore_signal(
            barrier_sem, inc=1,
            device_id=((my + o) % N_DEV,),
            device_id_type=pl.DeviceIdType.MESH,
        )
    pl.semaphore_wait(barrier_sem, N_DEV - 1)

    def compute_block(b):
        xb = xb_ref[pl.ds(b * BLK, BLK), :]
        qc = jnp.dot(xb, wqb_ref[...], preferred_element_type=jnp.float32)
        qc = (qc * SCALE_LOG2E).astype(jnp.bfloat16)
        outs = []
        for h in range(H_LOC):
            g = h // 4
            qh = qc[:, h * DH:(h + 1) * DH]
            s = lax.dot_general(
                qh, kb_ref[:, g, :], (((1,), (1,)), ((), ())),
                preferred_element_type=jnp.float32,
            )
            p = jnp.exp2(s).astype(jnp.bfloat16)
            pv = jnp.dot(p, vb_ref[:, g, :],
                         preferred_element_type=jnp.float32)
            o = pv[:, :DH] / pv[:, DH:DH + 1]
            outs.append(o.astype(jnp.bfloat16))
        ab = jnp.concatenate(outs, axis=1)
        return jnp.dot(ab, wob_ref[...], preferred_element_type=jnp.float32)

    rs_rdmas = []

    def send_chunk(val, j):
        slot = (j - my) % N_DEV - 1
        stage_ref[slot, :, :] = val.astype(jnp.bfloat16)
        rdma = pltpu.make_async_remote_copy(
            src_ref=stage_ref.at[slot],
            dst_ref=comm_ref.at[slot],
            send_sem=rs_send_sems.at[slot],
            recv_sem=rs_recv_sems.at[slot],
            device_id=(j,),
            device_id_type=pl.DeviceIdType.MESH,
        )
        rdma.start()
        rs_rdmas.append(rdma)

    my_blk = my // 2
    part_my = None
    for t in range(NBLK):
        b = (my_blk + 1 + t) % NBLK
        part_blk = compute_block(b)
        if t < NBLK - 1:
            for half in range(2):
                send_chunk(part_blk[half * CHUNK:(half + 1) * CHUNK, :],
                           2 * b + half)
        else:
            pblk_ref[...] = part_blk
            mine_off = (my % 2) * CHUNK
            part_my = pblk_ref[pl.ds(mine_off, CHUNK), :]
            other = pblk_ref[pl.ds(CHUNK - mine_off, CHUNK), :]
            send_chunk(other, lax.bitwise_xor(my, 1))

    for rdma in rs_rdmas:
        rdma.wait()
    red = part_my + jnp.sum(comm_ref[...].astype(jnp.float32), axis=0)

    myrows = pl.ds(my * CHUNK, CHUNK)
    gbuf_ref[myrows, :] = red.astype(jnp.bfloat16)
    ag_rdmas = []
    for o in range(1, N_DEV):
        rdma = pltpu.make_async_remote_copy(
            src_ref=gbuf_ref.at[myrows, :],
            dst_ref=gbuf_ref.at[myrows, :],
            send_sem=ag_send_sems.at[o - 1],
            recv_sem=ag_recv_sems.at[o - 1],
            device_id=((my + o) % N_DEV,),
            device_id_type=pl.DeviceIdType.MESH,
        )
        rdma.start()
        ag_rdmas.append(rdma)

    out_ref[0, myrows, :] = red
    for o, rdma in enumerate(ag_rdmas, start=1):
        rdma.wait()
        rows = pl.ds(((my - o) % N_DEV) * CHUNK, CHUNK)
        out_ref[0, rows, :] = gbuf_ref[rows, :].astype(jnp.float32)


def kernel(x, Wq, Wo, K_ext, V_ext):
    bf = jnp.bfloat16

    return pl.pallas_call(
        _body,
        out_shape=jax.ShapeDtypeStruct((1, SQ, D), jnp.float32),
        in_specs=[
            pl.BlockSpec(memory_space=pltpu.VMEM),
            pl.BlockSpec(memory_space=pltpu.VMEM),
            pl.BlockSpec(memory_space=pltpu.VMEM),
            pl.BlockSpec(memory_space=pl.ANY),
            pl.BlockSpec(memory_space=pl.ANY),
        ],
        out_specs=pl.BlockSpec(memory_space=pltpu.VMEM),
        scratch_shapes=[
            pltpu.VMEM((SQ, D), bf),
            pltpu.VMEM((D, D), bf),
            pltpu.VMEM((D, D), bf),
            pltpu.VMEM((SKV, G_LOC, DH), jnp.float32),
            pltpu.VMEM((SKV, G_LOC, DH), jnp.float32),
            pltpu.VMEM((SKV, G_LOC, DH), bf),
            pltpu.VMEM((SKV, G_LOC, VAUG), bf),
            pltpu.VMEM((N_DEV - 1, CHUNK, D), bf),
            pltpu.VMEM((N_DEV - 1, CHUNK, D), bf),
            pltpu.VMEM((SQ, D), bf),
            pltpu.VMEM((BLK, D), jnp.float32),
            pltpu.SemaphoreType.DMA((2,)),
            pltpu.SemaphoreType.DMA((N_DEV - 1,)),
            pltpu.SemaphoreType.DMA((N_DEV - 1,)),
            pltpu.SemaphoreType.DMA((N_DEV - 1,)),
            pltpu.SemaphoreType.DMA((N_DEV - 1,)),
        ],
        compiler_params=pltpu.CompilerParams(
            collective_id=0,
            vmem_limit_bytes=100 * 1024 * 1024,
        ),
    )(x[0], Wq, Wo, K_ext.reshape(SKV, 16, DH), V_ext.reshape(SKV, 16, DH))


# device time: 57768 ns/iter; 1.2136x vs baseline; 1.2136x over previous
import jax
import jax.numpy as jnp
from jax import lax
from jax.experimental import pallas as pl
from jax.experimental.pallas import tpu as pltpu

N_DEV = 8
SQ = 512
D = 1024
DH = 128
H_LOC = 8
G_LOC = 2
SKV = 2048
SCALE = 0.08838834764831843
SCALE_LOG2E = SCALE * 1.4426950408889634
VAUG = 256
CHUNK = SQ // N_DEV
BLK = 128
NBLK = SQ // BLK


def _body(
    x_ref, wq_ref, wo_ref, kx_ref, vx_ref, out_ref,
    xb_ref, wqb_ref, wob_ref, kf_ref, vf_ref, kb_ref, vb_ref,
    qs_ref, ab_ref, stage_ref, comm_ref, gbuf_ref, pblk_ref,
    kv_sems, rs_send_sems, rs_recv_sems, ag_send_sems, ag_recv_sems,
):
    my = lax.axis_index("i")

    kcopy = pltpu.make_async_copy(
        kx_ref.at[:, pl.ds(2 * my, G_LOC), :], kf_ref, kv_sems.at[0])
    vcopy = pltpu.make_async_copy(
        vx_ref.at[:, pl.ds(2 * my, G_LOC), :], vf_ref, kv_sems.at[1])
    kcopy.start()
    vcopy.start()

    xb_ref[...] = x_ref[...].astype(jnp.bfloat16)
    wqb_ref[...] = wq_ref[...].astype(jnp.bfloat16)
    wob_ref[...] = wo_ref[...].astype(jnp.bfloat16)
    kcopy.wait()
    kb_ref[...] = kf_ref[...].astype(jnp.bfloat16)
    vcopy.wait()
    vb_ref[...] = vf_ref[...].astype(jnp.bfloat16)

    barrier_sem = pltpu.get_barrier_semaphore()
    for o in range(1, N_DEV):
        pl.semaphore_signal(
            barrier_sem, inc=1,
            device_id=((my + o) % N_DEV,),
            device_id_type=pl.DeviceIdType.MESH,
        )
    pl.semaphore_wait(barrier_sem, N_DEV - 1)

    def compute_block(b):
        xb = xb_ref[pl.ds(b * BLK, BLK), :]
        qc = jnp.dot(xb, wqb_ref[...], preferred_element_type=jnp.float32)
        qc = (qc * SCALE_LOG2E).astype(jnp.bfloat16)
        for g in range(G_LOC):
            for h in range(4):
                qs_ref[h * BLK:(h + 1) * BLK, :] = (
                    qc[:, (4 * g + h) * DH:(4 * g + h + 1) * DH])
            s = lax.dot_general(
                qs_ref[...], kb_ref[:, g, :], (((1,), (1,)), ((), ())),
                preferred_element_type=jnp.float32,
            )
            pf = jnp.exp2(s)
            p = pf.astype(jnp.bfloat16)
            l = jnp.sum(pf, axis=1, keepdims=True)
            pv = jnp.dot(p, vb_ref[:, g, :],
                         preferred_element_type=jnp.float32)
            o = (pv / l).astype(jnp.bfloat16)
            for h in range(4):
                ab_ref[:, (4 * g + h) * DH:(4 * g + h + 1) * DH] = (
                    o[h * BLK:(h + 1) * BLK, :])
        return jnp.dot(ab_ref[...], wob_ref[...],
                       preferred_element_type=jnp.float32)

    rs_rdmas = []

    def send_chunk(val, j):
        slot = (j - my) % N_DEV - 1
        stage_ref[slot, :, :] = val.astype(jnp.bfloat16)
        rdma = pltpu.make_async_remote_copy(
            src_ref=stage_ref.at[slot],
            dst_ref=comm_ref.at[slot],
            send_sem=rs_send_sems.at[slot],
            recv_sem=rs_recv_sems.at[slot],
            device_id=(j,),
            device_id_type=pl.DeviceIdType.MESH,
        )
        rdma.start()
        rs_rdmas.append(rdma)

    my_blk = my // 2
    part_my = None
    for t in range(NBLK):
        b = (my_blk + 1 + t) % NBLK
        part_blk = compute_block(b)
        if t < NBLK - 1:
            for half in range(2):
                send_chunk(part_blk[half * CHUNK:(half + 1) * CHUNK, :],
                           2 * b + half)
        else:
            pblk_ref[...] = part_blk
            mine_off = (my % 2) * CHUNK
            part_my = pblk_ref[pl.ds(mine_off, CHUNK), :]
            other = pblk_ref[pl.ds(CHUNK - mine_off, CHUNK), :]
            send_chunk(other, lax.bitwise_xor(my, 1))

    for rdma in rs_rdmas:
        rdma.wait()
    red = part_my + jnp.sum(comm_ref[...].astype(jnp.float32), axis=0)

    myrows = pl.ds(my * CHUNK, CHUNK)
    gbuf_ref[myrows, :] = red.astype(jnp.bfloat16)
    ag_rdmas = []
    for o in range(1, N_DEV):
        rdma = pltpu.make_async_remote_copy(
            src_ref=gbuf_ref.at[myrows, :],
            dst_ref=gbuf_ref.at[myrows, :],
            send_sem=ag_send_sems.at[o - 1],
            recv_sem=ag_recv_sems.at[o - 1],
            device_id=((my + o) % N_DEV,),
            device_id_type=pl.DeviceIdType.MESH,
        )
        rdma.start()
        ag_rdmas.append(rdma)

    out_ref[0, myrows, :] = red
    for o, rdma in enumerate(ag_rdmas, start=1):
        rdma.wait()
        rows = pl.ds(((my - o) % N_DEV) * CHUNK, CHUNK)
        out_ref[0, rows, :] = gbuf_ref[rows, :].astype(jnp.float32)


def kernel(x, Wq, Wo, K_ext, V_ext):
    bf = jnp.bfloat16

    return pl.pallas_call(
        _body,
        out_shape=jax.ShapeDtypeStruct((1, SQ, D), jnp.float32),
        in_specs=[
            pl.BlockSpec(memory_space=pltpu.VMEM),
            pl.BlockSpec(memory_space=pltpu.VMEM),
            pl.BlockSpec(memory_space=pltpu.VMEM),
            pl.BlockSpec(memory_space=pl.ANY),
            pl.BlockSpec(memory_space=pl.ANY),
        ],
        out_specs=pl.BlockSpec(memory_space=pltpu.VMEM),
        scratch_shapes=[
            pltpu.VMEM((SQ, D), bf),
            pltpu.VMEM((D, D), bf),
            pltpu.VMEM((D, D), bf),
            pltpu.VMEM((SKV, G_LOC, DH), jnp.float32),
            pltpu.VMEM((SKV, G_LOC, DH), jnp.float32),
            pltpu.VMEM((SKV, G_LOC, DH), bf),
            pltpu.VMEM((SKV, G_LOC, DH), bf),
            pltpu.VMEM((4 * BLK, DH), bf),
            pltpu.VMEM((BLK, D), bf),
            pltpu.VMEM((N_DEV - 1, CHUNK, D), bf),
            pltpu.VMEM((N_DEV - 1, CHUNK, D), bf),
            pltpu.VMEM((SQ, D), bf),
            pltpu.VMEM((BLK, D), jnp.float32),
            pltpu.SemaphoreType.DMA((2,)),
            pltpu.SemaphoreType.DMA((N_DEV - 1,)),
            pltpu.SemaphoreType.DMA((N_DEV - 1,)),
            pltpu.SemaphoreType.DMA((N_DEV - 1,)),
            pltpu.SemaphoreType.DMA((N_DEV - 1,)),
        ],
        compiler_params=pltpu.CompilerParams(
            collective_id=0,
            vmem_limit_bytes=100 * 1024 * 1024,
        ),
    )(x[0], Wq, Wo, K_ext.reshape(SKV, 16, DH), V_ext.reshape(SKV, 16, DH))


# device time: 50350 ns/iter; 1.3924x vs baseline; 1.1473x over previous
import jax
import jax.numpy as jnp
from jax import lax
from jax.experimental import pallas as pl
from jax.experimental.pallas import tpu as pltpu

N_DEV = 8
SQ = 512
D = 1024
DH = 128
H_LOC = 8
G_LOC = 2
SKV = 2048
SCALE = 0.08838834764831843
SCALE_LOG2E = SCALE * 1.4426950408889634
VAUG = 256
CHUNK = SQ // N_DEV
BLK = 256
NBLK = SQ // BLK
CPB = BLK // CHUNK


def _body(
    x_ref, wq_ref, wo_ref, kx_ref, vx_ref, out_ref,
    xb_ref, wqb_ref, wob_ref, kf_ref, vf_ref, kb_ref, vb_ref,
    qs_ref, ab_ref, stage_ref, comm_ref, gbuf_ref, pblk_ref,
    kv_sems, rs_send_sems, rs_recv_sems, ag_send_sems, ag_recv_sems,
):
    my = lax.axis_index("i")

    kcopy = pltpu.make_async_copy(
        kx_ref.at[:, pl.ds(2 * my, G_LOC), :], kf_ref, kv_sems.at[0])
    vcopy = pltpu.make_async_copy(
        vx_ref.at[:, pl.ds(2 * my, G_LOC), :], vf_ref, kv_sems.at[1])
    kcopy.start()
    vcopy.start()

    xb_ref[...] = x_ref[...].astype(jnp.bfloat16)
    wqb_ref[...] = wq_ref[...].astype(jnp.bfloat16)
    wob_ref[...] = wo_ref[...].astype(jnp.bfloat16)
    kcopy.wait()
    kb_ref[...] = kf_ref[...].astype(jnp.bfloat16)
    vcopy.wait()
    vb_ref[...] = vf_ref[...].astype(jnp.bfloat16)

    barrier_sem = pltpu.get_barrier_semaphore()
    for o in range(1, N_DEV):
        pl.semaphore_signal(
            barrier_sem, inc=1,
            device_id=((my + o) % N_DEV,),
            device_id_type=pl.DeviceIdType.MESH,
        )
    pl.semaphore_wait(barrier_sem, N_DEV - 1)

    def compute_block(b):
        xb = xb_ref[pl.ds(b * BLK, BLK), :]
        qc = jnp.dot(xb, wqb_ref[...], preferred_element_type=jnp.float32)
        qc = (qc * SCALE_LOG2E).astype(jnp.bfloat16)
        for g in range(G_LOC):
            for h in range(4):
                qs_ref[h * BLK:(h + 1) * BLK, :] = (
                    qc[:, (4 * g + h) * DH:(4 * g + h + 1) * DH])
            s = lax.dot_general(
                qs_ref[...], kb_ref[:, g, :], (((1,), (1,)), ((), ())),
                preferred_element_type=jnp.float32,
            )
            pf = jnp.exp2(s)
            p = pf.astype(jnp.bfloat16)
            l = jnp.sum(pf, axis=1, keepdims=True)
            pv = jnp.dot(p, vb_ref[:, g, :],
                         preferred_element_type=jnp.float32)
            o = (pv / l).astype(jnp.bfloat16)
            for h in range(4):
                ab_ref[:, (4 * g + h) * DH:(4 * g + h + 1) * DH] = (
                    o[h * BLK:(h + 1) * BLK, :])
        return jnp.dot(ab_ref[...], wob_ref[...],
                       preferred_element_type=jnp.float32)

    rs_rdmas = []

    def send_chunk(val, j):
        slot = (j - my) % N_DEV - 1
        stage_ref[slot, :, :] = val.astype(jnp.bfloat16)
        rdma = pltpu.make_async_remote_copy(
            src_ref=stage_ref.at[slot],
            dst_ref=comm_ref.at[slot],
            send_sem=rs_send_sems.at[slot],
            recv_sem=rs_recv_sems.at[slot],
            device_id=(j,),
            device_id_type=pl.DeviceIdType.MESH,
        )
        rdma.start()
        rs_rdmas.append(rdma)

    my_blk = my // CPB
    part_my = None
    for t in range(NBLK):
        b = (my_blk + 1 + t) % NBLK
        part_blk = compute_block(b)
        if t < NBLK - 1:
            for q in range(CPB):
                send_chunk(part_blk[q * CHUNK:(q + 1) * CHUNK, :],
                           CPB * b + q)
        else:
            pblk_ref[...] = part_blk
            qmy = my % CPB
            part_my = pblk_ref[pl.ds(qmy * CHUNK, CHUNK), :]
            for d in range(1, CPB):
                qd = (qmy + d) % CPB
                send_chunk(pblk_ref[pl.ds(qd * CHUNK, CHUNK), :],
                           CPB * b + qd)

    for rdma in rs_rdmas:
        rdma.wait()
    red = part_my + jnp.sum(comm_ref[...].astype(jnp.float32), axis=0)

    myrows = pl.ds(my * CHUNK, CHUNK)
    gbuf_ref[myrows, :] = red.astype(jnp.bfloat16)
    ag_rdmas = []
    for o in range(1, N_DEV):
        rdma = pltpu.make_async_remote_copy(
            src_ref=gbuf_ref.at[myrows, :],
            dst_ref=gbuf_ref.at[myrows, :],
            send_sem=ag_send_sems.at[o - 1],
            recv_sem=ag_recv_sems.at[o - 1],
            device_id=((my + o) % N_DEV,),
            device_id_type=pl.DeviceIdType.MESH,
        )
        rdma.start()
        ag_rdmas.append(rdma)

    out_ref[0, myrows, :] = red
    for o, rdma in enumerate(ag_rdmas, start=1):
        rdma.wait()
        rows = pl.ds(((my - o) % N_DEV) * CHUNK, CHUNK)
        out_ref[0, rows, :] = gbuf_ref[rows, :].astype(jnp.float32)


def kernel(x, Wq, Wo, K_ext, V_ext):
    bf = jnp.bfloat16

    return pl.pallas_call(
        _body,
        out_shape=jax.ShapeDtypeStruct((1, SQ, D), jnp.float32),
        in_specs=[
            pl.BlockSpec(memory_space=pltpu.VMEM),
            pl.BlockSpec(memory_space=pltpu.VMEM),
            pl.BlockSpec(memory_space=pltpu.VMEM),
            pl.BlockSpec(memory_space=pl.ANY),
            pl.BlockSpec(memory_space=pl.ANY),
        ],
        out_specs=pl.BlockSpec(memory_space=pltpu.VMEM),
        scratch_shapes=[
            pltpu.VMEM((SQ, D), bf),
            pltpu.VMEM((D, D), bf),
            pltpu.VMEM((D, D), bf),
            pltpu.VMEM((SKV, G_LOC, DH), jnp.float32),
            pltpu.VMEM((SKV, G_LOC, DH), jnp.float32),
            pltpu.VMEM((SKV, G_LOC, DH), bf),
            pltpu.VMEM((SKV, G_LOC, DH), bf),
            pltpu.VMEM((4 * BLK, DH), bf),
            pltpu.VMEM((BLK, D), bf),
            pltpu.VMEM((N_DEV - 1, CHUNK, D), bf),
            pltpu.VMEM((N_DEV - 1, CHUNK, D), bf),
            pltpu.VMEM((SQ, D), bf),
            pltpu.VMEM((BLK, D), jnp.float32),
            pltpu.SemaphoreType.DMA((2,)),
            pltpu.SemaphoreType.DMA((N_DEV - 1,)),
            pltpu.SemaphoreType.DMA((N_DEV - 1,)),
            pltpu.SemaphoreType.DMA((N_DEV - 1,)),
            pltpu.SemaphoreType.DMA((N_DEV - 1,)),
        ],
        compiler_params=pltpu.CompilerParams(
            collective_id=0,
            vmem_limit_bytes=100 * 1024 * 1024,
        ),
    )(x[0], Wq, Wo, K_ext.reshape(SKV, 16, DH), V_ext.reshape(SKV, 16, DH))
